# baseline (speedup 1.0000x reference)
import functools

import jax
import jax.numpy as jnp
from jax import lax
from jax.experimental import pallas as pl
from jax.experimental.pallas import tpu as pltpu
from jax.experimental.pallas import tpu_sc as plsc

N_ATOMS = 320000
N_GRAPHS = 10000
FEAT = 128
HIDDEN = 100

_NC = 2
_NS = 16
_NW = _NC * _NS
_B = 128
_NB = N_ATOMS // _B
_SPLIT_BLK = 2024
_NTRI = (_SPLIT_BLK // _NW + 3) // 3
_RPT = 624
_WCH = (128, 128, 128, 128, 112)

_W = 256
_NWIN = (N_GRAPHS + _W - 1) // _W
_CHK = 512
_NCHK = N_ATOMS // _CHK
_SPLIT_CHUNK = _SPLIT_BLK * _B // _CHK


def _make_segsum():
    mesh = plsc.VectorSubcoreMesh(core_axis_name="c", subcore_axis_name="s")

    @functools.partial(
        pl.kernel,
        mesh=mesh,
        out_type=jax.ShapeDtypeStruct((_NC * N_GRAPHS, FEAT), jnp.float32),
        scratch_types=[
            pltpu.VMEM((3, _B, FEAT), jnp.float32),
            pltpu.VMEM((3, 1, _B), jnp.int32),
            pltpu.SemaphoreType.DMA,
            pltpu.SemaphoreType.DMA,
            pltpu.SemaphoreType.DMA,
            pltpu.VMEM_SHARED((N_GRAPHS, FEAT), jnp.float32),
        ],
    )
    def segsum(af_hbm, mem_hbm, out_hbm, rows_v, idx_v, sem0, sem1, sem2,
               acc_sh):
        c = lax.axis_index("c")
        s = lax.axis_index("s")
        wid = c * _NS + s
        sems = (sem0, sem1, sem2)
        r0 = s * _RPT
        ob = c * N_GRAPHS + r0
        last = s == _NS - 1

        def rslice(b, n, m=_B):
            return rows_v.at[b] if n == m else rows_v.at[b, pl.ds(0, n)]

        def issue(k, b):
            g = wid + k * _NW

            @pl.when(g < _SPLIT_BLK)
            def _():
                pltpu.async_copy(af_hbm.at[pl.ds(g * _B, _B)],
                                 rows_v.at[b], sems[b])
                pltpu.async_copy(mem_hbm.at[g], idx_v.at[b], sems[b])

        def consume(k, b):
            g = wid + k * _NW

            @pl.when(g < _SPLIT_BLK)
            def _():
                pltpu.make_async_copy(af_hbm.at[pl.ds(g * _B, _B)],
                                      rows_v.at[b], sems[b]).wait()
                pltpu.make_async_copy(mem_hbm.at[g], idx_v.at[b],
                                      sems[b]).wait()
                pltpu.sync_copy(rows_v.at[b], acc_sh.at[idx_v.at[b, 0]],
                                add=True)

            issue(k + 3, b)

        issue(0, 0)
        issue(1, 1)

        zero16 = jnp.zeros((16,), jnp.float32)

        def zbody(i, carry):
            for j in range(FEAT // 16):
                rows_v[2, i, pl.ds(j * 16, 16)] = zero16
            return carry

        lax.fori_loop(0, _B, zbody, 0)

        off = 0
        for n in _WCH:
            pltpu.async_copy(rslice(2, n), acc_sh.at[pl.ds(r0 + off, n)],
                             sem2)
            off += n

        @pl.when(last)
        def _():
            pltpu.async_copy(rslice(2, 16), acc_sh.at[pl.ds(r0 + 624, 16)],
                             sem2)

        off = 0
        for n in _WCH:
            pltpu.make_async_copy(rslice(2, n),
                                  acc_sh.at[pl.ds(r0 + off, n)], sem2).wait()
            off += n

        @pl.when(last)
        def _():
            pltpu.make_async_copy(rslice(2, 16),
                                  acc_sh.at[pl.ds(r0 + 624, 16)],
                                  sem2).wait()

        issue(2, 2)
        plsc.subcore_barrier()

        def tri(kk, carry):
            k = kk * 3
            consume(k, 0)
            consume(k + 1, 1)
            consume(k + 2, 2)
            return carry

        lax.fori_loop(0, _NTRI, tri, 0)
        plsc.subcore_barrier()

        def st_dsc(z, n):
            b = z % 2
            return (rslice(b, n),
                    out_hbm.at[pl.ds(ob + z * _B, n)], sems[b])

        for z, n in enumerate(_WCH):
            if z >= 2:
                src, dst, sem = st_dsc(z - 2, _WCH[z - 2])
                pltpu.make_async_copy(src, dst, sem).wait()
            src, dst, sem = st_dsc(z, n)
            pltpu.sync_copy(acc_sh.at[pl.ds(r0 + z * _B, n)], rslice(z % 2, n))
            pltpu.async_copy(src, dst, sem)
        for z in (3, 4):
            src, dst, sem = st_dsc(z, _WCH[z])
            pltpu.make_async_copy(src, dst, sem).wait()

        @pl.when(last)
        def _():
            pltpu.sync_copy(acc_sh.at[pl.ds(r0 + 624, 16)], rslice(0, 16))
            pltpu.sync_copy(rslice(0, 16), out_hbm.at[pl.ds(ob + 624, 16)])

    return segsum


_segsum = _make_segsum()


def _tc_body(lo_ref, hi_ref, af_ref, mem_ref, out_ref, rows_s, mem_s,
             sem0, sem1):
    w = pl.program_id(0)
    lo = lo_ref[w]
    hi = hi_ref[w]
    n = hi - lo
    out_ref[...] = jnp.zeros((_W, FEAT), jnp.float32)
    iot = lax.broadcasted_iota(jnp.int32, (_W, _CHK), 0) + w * _W
    sems = (sem0, sem1)

    def dscs(j, b):
        c = lo + j
        return (pltpu.make_async_copy(af_ref.at[pl.ds(c * _CHK, _CHK)],
                                      rows_s.at[b], sems[b]),
                pltpu.make_async_copy(mem_ref.at[c], mem_s.at[b], sems[b]))

    def issue(j, b, guard):
        @pl.when(guard)
        def _():
            cp_r, cp_m = dscs(j, b)
            cp_r.start()
            cp_m.start()

    def step(j, b, guard):
        @pl.when(guard)
        def _():
            cp_r, cp_m = dscs(j, b)
            cp_r.wait()
            cp_m.wait()
            oh = (iot == mem_s[b]).astype(jnp.bfloat16)
            rows = rows_s[b]
            r_hi = rows.astype(jnp.bfloat16)
            r_lo = (rows - r_hi.astype(jnp.float32)).astype(jnp.bfloat16)
            dn = (((1,), (0,)), ((), ()))
            out_ref[...] += (
                lax.dot_general(oh, r_hi, dn,
                                preferred_element_type=jnp.float32)
                + lax.dot_general(oh, r_lo, dn,
                                  preferred_element_type=jnp.float32))

    issue(0, 0, n > 0)

    def pair(kk, carry):
        k = kk * 2
        issue(k + 1, 1, k + 1 < n)
        step(k, 0, k < n)
        issue(k + 2, 0, k + 2 < n)
        step(k + 1, 1, k + 1 < n)
        return carry

    lax.fori_loop(0, (n + 1) // 2, pair, 0)


def _tc_segsum(atom_features, mem3, lo_c, hi_c):
    grid_spec = pltpu.PrefetchScalarGridSpec(
        num_scalar_prefetch=2,
        grid=(_NWIN,),
        in_specs=[
            pl.BlockSpec(memory_space=pl.ANY),
            pl.BlockSpec(memory_space=pl.ANY),
        ],
        out_specs=pl.BlockSpec((_W, FEAT), lambda i, lo, hi: (i, 0)),
        scratch_shapes=[
            pltpu.VMEM((2, _CHK, FEAT), jnp.float32),
            pltpu.VMEM((2, 1, _CHK), jnp.int32),
            pltpu.SemaphoreType.DMA,
            pltpu.SemaphoreType.DMA,
        ],
    )
    return pl.pallas_call(
        _tc_body,
        grid_spec=grid_spec,
        out_shape=jax.ShapeDtypeStruct((_NWIN * _W, FEAT), jnp.float32),
    )(lo_c, hi_c, atom_features, mem3)


_MLP_BLK = 1000
_MLP_GRID = N_GRAPHS // _MLP_BLK


def _mlp_body(*refs):
    if len(refs) == 8:
        p0_ref, p1_ref, p2_ref, w1_ref, b1_ref, w2_ref, b2_ref, o_ref = refs
        g = p0_ref[...] + p1_ref[...] + p2_ref[...]
    else:
        p0_ref, p1_ref, w1_ref, b1_ref, w2_ref, b2_ref, o_ref = refs
        g = p0_ref[...] + p1_ref[...]
    h = jnp.dot(g, w1_ref[...], preferred_element_type=jnp.float32)
    h = jnp.maximum(h + b1_ref[...], 0.0)
    o = jnp.dot(h, w2_ref[...], preferred_element_type=jnp.float32)
    o_ref[...] = jnp.maximum(o + b2_ref[...], 0.0)


def _mlp(partials, p_tc, W1, b1, W2, b2):
    in_specs = [
        pl.BlockSpec((_MLP_BLK, FEAT), lambda i: (i, 0)),
        pl.BlockSpec((_MLP_BLK, FEAT), lambda i: (i + _MLP_GRID, 0)),
    ]
    args = [partials, partials]
    if p_tc is not None:
        in_specs.append(pl.BlockSpec((_MLP_BLK, FEAT), lambda i: (i, 0)))
        args.append(p_tc)
    in_specs += [
        pl.BlockSpec((FEAT, HIDDEN), lambda i: (0, 0)),
        pl.BlockSpec((1, HIDDEN), lambda i: (0, 0)),
        pl.BlockSpec((HIDDEN, FEAT), lambda i: (0, 0)),
        pl.BlockSpec((1, FEAT), lambda i: (0, 0)),
    ]
    args += [W1, b1.reshape(1, HIDDEN), W2, b2.reshape(1, FEAT)]
    return pl.pallas_call(
        _mlp_body,
        grid=(_MLP_GRID,),
        in_specs=in_specs,
        out_specs=pl.BlockSpec((_MLP_BLK, FEAT), lambda i: (i, 0)),
        out_shape=jax.ShapeDtypeStruct((N_GRAPHS, FEAT), jnp.float32),
    )(*args)


def kernel(atom_features, membership, W1, b1, W2, b2):
    mem_i32 = membership.astype(jnp.int32)
    mem_sc = mem_i32.reshape(_NB, 1, _B)

    if _SPLIT_BLK < _NB:
        mem_tc = mem_i32.reshape(_NCHK, 1, _CHK)
        seg_bounds = jnp.arange(_NWIN + 1, dtype=jnp.int32) * _W
        pos = jnp.searchsorted(mem_i32, seg_bounds,
                               side="left").astype(jnp.int32)
        lo_c = jnp.maximum(pos[:-1] // _CHK, _SPLIT_CHUNK)
        hi_c = jnp.minimum((pos[1:] + _CHK - 1) // _CHK, _NCHK)
        hi_c = jnp.maximum(hi_c, lo_c)
        partials = _segsum(atom_features, mem_sc)
        p_tc = _tc_segsum(atom_features, mem_tc, lo_c, hi_c)
    else:
        partials = _segsum(atom_features, mem_sc)
        p_tc = None

    return _mlp(partials, p_tc, W1, b1, W2, b2)


# speedup vs baseline: 1.1326x; 1.1326x over previous
import functools

import jax
import jax.numpy as jnp
from jax import lax
from jax.experimental import pallas as pl
from jax.experimental.pallas import tpu as pltpu
from jax.experimental.pallas import tpu_sc as plsc

N_ATOMS = 320000
N_GRAPHS = 10000
FEAT = 128
HIDDEN = 100

_NC = 2
_NS = 16
_NW = _NC * _NS
_B = 128
_NB = N_ATOMS // _B
_SPLIT_BLK = 2500
_NTRI = (_SPLIT_BLK // _NW + 3) // 3
_RPT = 624
_WCH = (128, 128, 128, 128, 112)

_W = 256
_NWIN = (N_GRAPHS + _W - 1) // _W
_CHK = 512
_NCHK = N_ATOMS // _CHK
_SPLIT_CHUNK = _SPLIT_BLK * _B // _CHK


def _make_segsum():
    mesh = plsc.VectorSubcoreMesh(core_axis_name="c", subcore_axis_name="s")

    @functools.partial(
        pl.kernel,
        mesh=mesh,
        out_type=jax.ShapeDtypeStruct((_NC * N_GRAPHS, FEAT), jnp.float32),
        scratch_types=[
            pltpu.VMEM((3, _B, FEAT), jnp.float32),
            pltpu.VMEM((3, 1, _B), jnp.int32),
            pltpu.SemaphoreType.DMA,
            pltpu.SemaphoreType.DMA,
            pltpu.SemaphoreType.DMA,
            pltpu.VMEM_SHARED((N_GRAPHS, FEAT), jnp.float32),
        ],
    )
    def segsum(af_hbm, mem_hbm, out_hbm, rows_v, idx_v, sem0, sem1, sem2,
               acc_sh):
        c = lax.axis_index("c")
        s = lax.axis_index("s")
        wid = c * _NS + s
        sems = (sem0, sem1, sem2)
        r0 = s * _RPT
        ob = c * N_GRAPHS + r0
        last = s == _NS - 1

        def rslice(b, n, m=_B):
            return rows_v.at[b] if n == m else rows_v.at[b, pl.ds(0, n)]

        def issue(k, b):
            g = wid + k * _NW

            @pl.when(g < _SPLIT_BLK)
            def _():
                pltpu.async_copy(af_hbm.at[pl.ds(g * _B, _B)],
                                 rows_v.at[b], sems[b])
                pltpu.async_copy(mem_hbm.at[g], idx_v.at[b], sems[b])

        def consume(k, b):
            g = wid + k * _NW

            @pl.when(g < _SPLIT_BLK)
            def _():
                pltpu.make_async_copy(af_hbm.at[pl.ds(g * _B, _B)],
                                      rows_v.at[b], sems[b]).wait()
                pltpu.make_async_copy(mem_hbm.at[g], idx_v.at[b],
                                      sems[b]).wait()
                pltpu.sync_copy(rows_v.at[b], acc_sh.at[idx_v.at[b, 0]],
                                add=True)

            issue(k + 3, b)

        issue(0, 0)
        issue(1, 1)

        zero16 = jnp.zeros((16,), jnp.float32)

        def zbody(i, carry):
            for j in range(FEAT // 16):
                rows_v[2, i, pl.ds(j * 16, 16)] = zero16
            return carry

        lax.fori_loop(0, _B, zbody, 0)

        off = 0
        for n in _WCH:
            pltpu.async_copy(rslice(2, n), acc_sh.at[pl.ds(r0 + off, n)],
                             sem2)
            off += n

        @pl.when(last)
        def _():
            pltpu.async_copy(rslice(2, 16), acc_sh.at[pl.ds(r0 + 624, 16)],
                             sem2)

        off = 0
        for n in _WCH:
            pltpu.make_async_copy(rslice(2, n),
                                  acc_sh.at[pl.ds(r0 + off, n)], sem2).wait()
            off += n

        @pl.when(last)
        def _():
            pltpu.make_async_copy(rslice(2, 16),
                                  acc_sh.at[pl.ds(r0 + 624, 16)],
                                  sem2).wait()

        issue(2, 2)
        plsc.subcore_barrier()

        def tri(kk, carry):
            k = kk * 3
            consume(k, 0)
            consume(k + 1, 1)
            consume(k + 2, 2)
            return carry

        lax.fori_loop(0, _NTRI, tri, 0)
        plsc.subcore_barrier()

        def st_dsc(z, n):
            b = z % 2
            return (rslice(b, n),
                    out_hbm.at[pl.ds(ob + z * _B, n)], sems[b])

        for z, n in enumerate(_WCH):
            if z >= 2:
                src, dst, sem = st_dsc(z - 2, _WCH[z - 2])
                pltpu.make_async_copy(src, dst, sem).wait()
            src, dst, sem = st_dsc(z, n)
            pltpu.sync_copy(acc_sh.at[pl.ds(r0 + z * _B, n)], rslice(z % 2, n))
            pltpu.async_copy(src, dst, sem)
        for z in (3, 4):
            src, dst, sem = st_dsc(z, _WCH[z])
            pltpu.make_async_copy(src, dst, sem).wait()

        @pl.when(last)
        def _():
            pltpu.sync_copy(acc_sh.at[pl.ds(r0 + 624, 16)], rslice(0, 16))
            pltpu.sync_copy(rslice(0, 16), out_hbm.at[pl.ds(ob + 624, 16)])

    return segsum


_segsum = _make_segsum()


def _tc_body(lo_ref, hi_ref, af_ref, mem_ref, out_ref, rows_s, mem_s,
             sem0, sem1):
    w = pl.program_id(0)
    lo = lo_ref[w]
    hi = hi_ref[w]
    n = hi - lo
    out_ref[...] = jnp.zeros((_W, FEAT), jnp.float32)
    iot = lax.broadcasted_iota(jnp.int32, (_W, _CHK), 0) + w * _W
    sems = (sem0, sem1)

    def dscs(j, b):
        c = lo + j
        return (pltpu.make_async_copy(af_ref.at[pl.ds(c * _CHK, _CHK)],
                                      rows_s.at[b], sems[b]),
                pltpu.make_async_copy(mem_ref.at[c], mem_s.at[b], sems[b]))

    def issue(j, b, guard):
        @pl.when(guard)
        def _():
            cp_r, cp_m = dscs(j, b)
            cp_r.start()
            cp_m.start()

    def step(j, b, guard):
        @pl.when(guard)
        def _():
            cp_r, cp_m = dscs(j, b)
            cp_r.wait()
            cp_m.wait()
            oh = (iot == mem_s[b]).astype(jnp.bfloat16)
            rows = rows_s[b]
            r_hi = rows.astype(jnp.bfloat16)
            r_lo = (rows - r_hi.astype(jnp.float32)).astype(jnp.bfloat16)
            dn = (((1,), (0,)), ((), ()))
            out_ref[...] += (
                lax.dot_general(oh, r_hi, dn,
                                preferred_element_type=jnp.float32)
                + lax.dot_general(oh, r_lo, dn,
                                  preferred_element_type=jnp.float32))

    issue(0, 0, n > 0)

    def pair(kk, carry):
        k = kk * 2
        issue(k + 1, 1, k + 1 < n)
        step(k, 0, k < n)
        issue(k + 2, 0, k + 2 < n)
        step(k + 1, 1, k + 1 < n)
        return carry

    lax.fori_loop(0, (n + 1) // 2, pair, 0)


def _tc_segsum(atom_features, mem3, lo_c, hi_c):
    grid_spec = pltpu.PrefetchScalarGridSpec(
        num_scalar_prefetch=2,
        grid=(_NWIN,),
        in_specs=[
            pl.BlockSpec(memory_space=pl.ANY),
            pl.BlockSpec(memory_space=pl.ANY),
        ],
        out_specs=pl.BlockSpec((_W, FEAT), lambda i, lo, hi: (i, 0)),
        scratch_shapes=[
            pltpu.VMEM((2, _CHK, FEAT), jnp.float32),
            pltpu.VMEM((2, 1, _CHK), jnp.int32),
            pltpu.SemaphoreType.DMA,
            pltpu.SemaphoreType.DMA,
        ],
    )
    return pl.pallas_call(
        _tc_body,
        grid_spec=grid_spec,
        out_shape=jax.ShapeDtypeStruct((_NWIN * _W, FEAT), jnp.float32),
    )(lo_c, hi_c, atom_features, mem3)


_MLP_BLK = 1000
_MLP_GRID = N_GRAPHS // _MLP_BLK


def _mlp_body(*refs):
    if len(refs) == 8:
        p0_ref, p1_ref, p2_ref, w1_ref, b1_ref, w2_ref, b2_ref, o_ref = refs
        g = p0_ref[...] + p1_ref[...] + p2_ref[...]
    else:
        p0_ref, p1_ref, w1_ref, b1_ref, w2_ref, b2_ref, o_ref = refs
        g = p0_ref[...] + p1_ref[...]
    h = jnp.dot(g, w1_ref[...], preferred_element_type=jnp.float32)
    h = jnp.maximum(h + b1_ref[...], 0.0)
    o = jnp.dot(h, w2_ref[...], preferred_element_type=jnp.float32)
    o_ref[...] = jnp.maximum(o + b2_ref[...], 0.0)


def _mlp(partials, p_tc, W1, b1, W2, b2):
    in_specs = [
        pl.BlockSpec((_MLP_BLK, FEAT), lambda i: (i, 0)),
        pl.BlockSpec((_MLP_BLK, FEAT), lambda i: (i + _MLP_GRID, 0)),
    ]
    args = [partials, partials]
    if p_tc is not None:
        in_specs.append(pl.BlockSpec((_MLP_BLK, FEAT), lambda i: (i, 0)))
        args.append(p_tc)
    in_specs += [
        pl.BlockSpec((FEAT, HIDDEN), lambda i: (0, 0)),
        pl.BlockSpec((1, HIDDEN), lambda i: (0, 0)),
        pl.BlockSpec((HIDDEN, FEAT), lambda i: (0, 0)),
        pl.BlockSpec((1, FEAT), lambda i: (0, 0)),
    ]
    args += [W1, b1.reshape(1, HIDDEN), W2, b2.reshape(1, FEAT)]
    return pl.pallas_call(
        _mlp_body,
        grid=(_MLP_GRID,),
        in_specs=in_specs,
        out_specs=pl.BlockSpec((_MLP_BLK, FEAT), lambda i: (i, 0)),
        out_shape=jax.ShapeDtypeStruct((N_GRAPHS, FEAT), jnp.float32),
    )(*args)


def kernel(atom_features, membership, W1, b1, W2, b2):
    mem_i32 = membership.astype(jnp.int32)
    mem_sc = mem_i32.reshape(_NB, 1, _B)

    if _SPLIT_BLK < _NB:
        mem_tc = mem_i32.reshape(_NCHK, 1, _CHK)
        seg_bounds = jnp.arange(_NWIN + 1, dtype=jnp.int32) * _W
        pos = jnp.searchsorted(mem_i32, seg_bounds,
                               side="left").astype(jnp.int32)
        lo_c = jnp.maximum(pos[:-1] // _CHK, _SPLIT_CHUNK)
        hi_c = jnp.minimum((pos[1:] + _CHK - 1) // _CHK, _NCHK)
        hi_c = jnp.maximum(hi_c, lo_c)
        partials = _segsum(atom_features, mem_sc)
        p_tc = _tc_segsum(atom_features, mem_tc, lo_c, hi_c)
    else:
        partials = _segsum(atom_features, mem_sc)
        p_tc = None

    return _mlp(partials, p_tc, W1, b1, W2, b2)


# speedup vs baseline: 1.2437x; 1.0981x over previous
import functools

import jax
import jax.numpy as jnp
from jax import lax
from jax.experimental import pallas as pl
from jax.experimental.pallas import tpu as pltpu
from jax.experimental.pallas import tpu_sc as plsc

N_ATOMS = 320000
N_GRAPHS = 10000
FEAT = 128
HIDDEN = 100

_NC = 2
_NS = 16
_NW = _NC * _NS
_B = 64
_NB = N_ATOMS // _B
_SPLIT_BLK = 5000
_NRING = 6
_NTRI = (_SPLIT_BLK // _NW + _NRING) // _NRING
_RPT = 624
_WCH = (64,) * 9 + (48,)

_W = 256
_NWIN = (N_GRAPHS + _W - 1) // _W
_CHK = 512
_NCHK = N_ATOMS // _CHK
_SPLIT_CHUNK = _SPLIT_BLK * _B // _CHK


def _make_segsum():
    mesh = plsc.VectorSubcoreMesh(core_axis_name="c", subcore_axis_name="s")

    @functools.partial(
        pl.kernel,
        mesh=mesh,
        out_type=jax.ShapeDtypeStruct((_NC * N_GRAPHS, FEAT), jnp.float32),
        scratch_types=[
            pltpu.VMEM((_NRING, _B, FEAT), jnp.float32),
            pltpu.VMEM((_NRING, 1, _B), jnp.int32),
            pltpu.SemaphoreType.DMA,
            pltpu.SemaphoreType.DMA,
            pltpu.SemaphoreType.DMA,
            pltpu.SemaphoreType.DMA,
            pltpu.SemaphoreType.DMA,
            pltpu.SemaphoreType.DMA,
            pltpu.VMEM_SHARED((N_GRAPHS, FEAT), jnp.float32),
        ],
    )
    def segsum(af_hbm, mem_hbm, out_hbm, rows_v, idx_v, sem0, sem1, sem2,
               sem3, sem4, sem5, acc_sh):
        c = lax.axis_index("c")
        s = lax.axis_index("s")
        wid = c * _NS + s
        sems = (sem0, sem1, sem2, sem3, sem4, sem5)
        r0 = s * _RPT
        ob = c * N_GRAPHS + r0
        last = s == _NS - 1

        def rslice(b, n, m=_B):
            return rows_v.at[b] if n == m else rows_v.at[b, pl.ds(0, n)]

        def issue(k, b):
            g = wid + k * _NW

            @pl.when(g < _SPLIT_BLK)
            def _():
                pltpu.async_copy(af_hbm.at[pl.ds(g * _B, _B)],
                                 rows_v.at[b], sems[b])
                pltpu.async_copy(mem_hbm.at[g], idx_v.at[b], sems[b])

        def consume(k, b):
            g = wid + k * _NW

            @pl.when(g < _SPLIT_BLK)
            def _():
                pltpu.make_async_copy(af_hbm.at[pl.ds(g * _B, _B)],
                                      rows_v.at[b], sems[b]).wait()
                pltpu.make_async_copy(mem_hbm.at[g], idx_v.at[b],
                                      sems[b]).wait()
                pltpu.sync_copy(rows_v.at[b], acc_sh.at[idx_v.at[b, 0]],
                                add=True)

            issue(k + _NRING, b)

        for _b in range(_NRING - 1):
            issue(_b, _b)

        zero16 = jnp.zeros((16,), jnp.float32)

        def zbody(i, carry):
            for j in range(FEAT // 16):
                rows_v[_NRING - 1, i, pl.ds(j * 16, 16)] = zero16
            return carry

        lax.fori_loop(0, _B, zbody, 0)

        zb = _NRING - 1
        off = 0
        for n in _WCH:
            pltpu.async_copy(rslice(zb, n), acc_sh.at[pl.ds(r0 + off, n)],
                             sems[zb])
            off += n

        @pl.when(last)
        def _():
            pltpu.async_copy(rslice(zb, 16), acc_sh.at[pl.ds(r0 + 624, 16)],
                             sems[zb])

        off = 0
        for n in _WCH:
            pltpu.make_async_copy(rslice(zb, n),
                                  acc_sh.at[pl.ds(r0 + off, n)],
                                  sems[zb]).wait()
            off += n

        @pl.when(last)
        def _():
            pltpu.make_async_copy(rslice(zb, 16),
                                  acc_sh.at[pl.ds(r0 + 624, 16)],
                                  sems[zb]).wait()

        issue(zb, zb)
        plsc.subcore_barrier()

        def tri(kk, carry):
            k = kk * _NRING
            for _b in range(_NRING):
                consume(k + _b, _b)
            return carry

        lax.fori_loop(0, _NTRI, tri, 0)
        plsc.subcore_barrier()

        def st_dsc(z, n):
            b = z % 2
            return (rslice(b, n),
                    out_hbm.at[pl.ds(ob + z * _B, n)], sems[b])

        for z, n in enumerate(_WCH):
            if z >= 2:
                src, dst, sem = st_dsc(z - 2, _WCH[z - 2])
                pltpu.make_async_copy(src, dst, sem).wait()
            src, dst, sem = st_dsc(z, n)
            pltpu.sync_copy(acc_sh.at[pl.ds(r0 + z * _B, n)], rslice(z % 2, n))
            pltpu.async_copy(src, dst, sem)
        for z in (len(_WCH) - 2, len(_WCH) - 1):
            src, dst, sem = st_dsc(z, _WCH[z])
            pltpu.make_async_copy(src, dst, sem).wait()

        @pl.when(last)
        def _():
            pltpu.sync_copy(acc_sh.at[pl.ds(r0 + 624, 16)], rslice(0, 16))
            pltpu.sync_copy(rslice(0, 16), out_hbm.at[pl.ds(ob + 624, 16)])

    return segsum


_segsum = _make_segsum()


def _tc_body(lo_ref, hi_ref, af_ref, mem_ref, out_ref, rows_s, mem_s,
             sem0, sem1):
    w = pl.program_id(0)
    lo = lo_ref[w]
    hi = hi_ref[w]
    n = hi - lo
    out_ref[...] = jnp.zeros((_W, FEAT), jnp.float32)
    iot = lax.broadcasted_iota(jnp.int32, (_W, _CHK), 0) + w * _W
    sems = (sem0, sem1)

    def dscs(j, b):
        c = lo + j
        return (pltpu.make_async_copy(af_ref.at[pl.ds(c * _CHK, _CHK)],
                                      rows_s.at[b], sems[b]),
                pltpu.make_async_copy(mem_ref.at[c], mem_s.at[b], sems[b]))

    def issue(j, b, guard):
        @pl.when(guard)
        def _():
            cp_r, cp_m = dscs(j, b)
            cp_r.start()
            cp_m.start()

    def step(j, b, guard):
        @pl.when(guard)
        def _():
            cp_r, cp_m = dscs(j, b)
            cp_r.wait()
            cp_m.wait()
            oh = (iot == mem_s[b]).astype(jnp.bfloat16)
            rows = rows_s[b]
            r_hi = rows.astype(jnp.bfloat16)
            r_lo = (rows - r_hi.astype(jnp.float32)).astype(jnp.bfloat16)
            dn = (((1,), (0,)), ((), ()))
            out_ref[...] += (
                lax.dot_general(oh, r_hi, dn,
                                preferred_element_type=jnp.float32)
                + lax.dot_general(oh, r_lo, dn,
                                  preferred_element_type=jnp.float32))

    issue(0, 0, n > 0)

    def pair(kk, carry):
        k = kk * 2
        issue(k + 1, 1, k + 1 < n)
        step(k, 0, k < n)
        issue(k + 2, 0, k + 2 < n)
        step(k + 1, 1, k + 1 < n)
        return carry

    lax.fori_loop(0, (n + 1) // 2, pair, 0)


def _tc_segsum(atom_features, mem3, lo_c, hi_c):
    grid_spec = pltpu.PrefetchScalarGridSpec(
        num_scalar_prefetch=2,
        grid=(_NWIN,),
        in_specs=[
            pl.BlockSpec(memory_space=pl.ANY),
            pl.BlockSpec(memory_space=pl.ANY),
        ],
        out_specs=pl.BlockSpec((_W, FEAT), lambda i, lo, hi: (i, 0)),
        scratch_shapes=[
            pltpu.VMEM((2, _CHK, FEAT), jnp.float32),
            pltpu.VMEM((2, 1, _CHK), jnp.int32),
            pltpu.SemaphoreType.DMA,
            pltpu.SemaphoreType.DMA,
        ],
    )
    return pl.pallas_call(
        _tc_body,
        grid_spec=grid_spec,
        out_shape=jax.ShapeDtypeStruct((_NWIN * _W, FEAT), jnp.float32),
    )(lo_c, hi_c, atom_features, mem3)


_MLP_BLK = 1000
_MLP_GRID = N_GRAPHS // _MLP_BLK


def _mlp_body(*refs):
    if len(refs) == 8:
        p0_ref, p1_ref, p2_ref, w1_ref, b1_ref, w2_ref, b2_ref, o_ref = refs
        g = p0_ref[...] + p1_ref[...] + p2_ref[...]
    else:
        p0_ref, p1_ref, w1_ref, b1_ref, w2_ref, b2_ref, o_ref = refs
        g = p0_ref[...] + p1_ref[...]
    h = jnp.dot(g, w1_ref[...], preferred_element_type=jnp.float32)
    h = jnp.maximum(h + b1_ref[...], 0.0)
    o = jnp.dot(h, w2_ref[...], preferred_element_type=jnp.float32)
    o_ref[...] = jnp.maximum(o + b2_ref[...], 0.0)


def _mlp(partials, p_tc, W1, b1, W2, b2):
    in_specs = [
        pl.BlockSpec((_MLP_BLK, FEAT), lambda i: (i, 0)),
        pl.BlockSpec((_MLP_BLK, FEAT), lambda i: (i + _MLP_GRID, 0)),
    ]
    args = [partials, partials]
    if p_tc is not None:
        in_specs.append(pl.BlockSpec((_MLP_BLK, FEAT), lambda i: (i, 0)))
        args.append(p_tc)
    in_specs += [
        pl.BlockSpec((FEAT, HIDDEN), lambda i: (0, 0)),
        pl.BlockSpec((1, HIDDEN), lambda i: (0, 0)),
        pl.BlockSpec((HIDDEN, FEAT), lambda i: (0, 0)),
        pl.BlockSpec((1, FEAT), lambda i: (0, 0)),
    ]
    args += [W1, b1.reshape(1, HIDDEN), W2, b2.reshape(1, FEAT)]
    return pl.pallas_call(
        _mlp_body,
        grid=(_MLP_GRID,),
        in_specs=in_specs,
        out_specs=pl.BlockSpec((_MLP_BLK, FEAT), lambda i: (i, 0)),
        out_shape=jax.ShapeDtypeStruct((N_GRAPHS, FEAT), jnp.float32),
    )(*args)


def kernel(atom_features, membership, W1, b1, W2, b2):
    mem_i32 = membership.astype(jnp.int32)
    mem_sc = mem_i32.reshape(_NB, 1, _B)

    if _SPLIT_BLK < _NB:
        mem_tc = mem_i32.reshape(_NCHK, 1, _CHK)
        seg_bounds = jnp.arange(_NWIN + 1, dtype=jnp.int32) * _W
        pos = jnp.searchsorted(mem_i32, seg_bounds,
                               side="left").astype(jnp.int32)
        lo_c = jnp.maximum(pos[:-1] // _CHK, _SPLIT_CHUNK)
        hi_c = jnp.minimum((pos[1:] + _CHK - 1) // _CHK, _NCHK)
        hi_c = jnp.maximum(hi_c, lo_c)
        partials = _segsum(atom_features, mem_sc)
        p_tc = _tc_segsum(atom_features, mem_tc, lo_c, hi_c)
    else:
        partials = _segsum(atom_features, mem_sc)
        p_tc = None

    return _mlp(partials, p_tc, W1, b1, W2, b2)


# speedup vs baseline: 1.2652x; 1.0173x over previous
import functools

import jax
import jax.numpy as jnp
from jax import lax
from jax.experimental import pallas as pl
from jax.experimental.pallas import tpu as pltpu
from jax.experimental.pallas import tpu_sc as plsc

N_ATOMS = 320000
N_GRAPHS = 10000
FEAT = 128
HIDDEN = 100

_NC = 2
_NS = 16
_NW = _NC * _NS
_B = 32
_NB = N_ATOMS // _B
_SPLIT_BLK = 10000
_NRING = 12
_NTRI = (_SPLIT_BLK // _NW + _NRING) // _NRING
_RPT = 624
_WCH = (32,) * 19 + (16,)

_W = 256
_NWIN = (N_GRAPHS + _W - 1) // _W
_CHK = 512
_NCHK = N_ATOMS // _CHK
_SPLIT_CHUNK = _SPLIT_BLK * _B // _CHK


def _make_segsum():
    mesh = plsc.VectorSubcoreMesh(core_axis_name="c", subcore_axis_name="s")

    @functools.partial(
        pl.kernel,
        mesh=mesh,
        out_type=jax.ShapeDtypeStruct((_NC * N_GRAPHS, FEAT), jnp.float32),
        scratch_types=[
            pltpu.VMEM((_NRING, _B, FEAT), jnp.float32),
            pltpu.VMEM((_NRING, 1, _B), jnp.int32),
            pltpu.SemaphoreType.DMA,
            pltpu.SemaphoreType.DMA,
            pltpu.SemaphoreType.DMA,
            pltpu.SemaphoreType.DMA,
            pltpu.SemaphoreType.DMA,
            pltpu.SemaphoreType.DMA,
            pltpu.SemaphoreType.DMA,
            pltpu.SemaphoreType.DMA,
            pltpu.SemaphoreType.DMA,
            pltpu.SemaphoreType.DMA,
            pltpu.SemaphoreType.DMA,
            pltpu.SemaphoreType.DMA,

            pltpu.VMEM_SHARED((N_GRAPHS, FEAT), jnp.float32),
        ],
    )
    def segsum(af_hbm, mem_hbm, out_hbm, rows_v, idx_v, sem0, sem1, sem2,
               sem3, sem4, sem5, sem6, sem7, sem8, sem9, sem10, sem11,
               acc_sh):
        c = lax.axis_index("c")
        s = lax.axis_index("s")
        wid = c * _NS + s
        sems = (sem0, sem1, sem2, sem3, sem4, sem5, sem6,
                sem7, sem8, sem9, sem10, sem11)
        r0 = s * _RPT
        ob = c * N_GRAPHS + r0
        last = s == _NS - 1

        def rslice(b, n, m=_B):
            return rows_v.at[b] if n == m else rows_v.at[b, pl.ds(0, n)]

        def issue(k, b):
            g = wid + k * _NW

            @pl.when(g < _SPLIT_BLK)
            def _():
                pltpu.async_copy(af_hbm.at[pl.ds(g * _B, _B)],
                                 rows_v.at[b], sems[b])
                pltpu.async_copy(mem_hbm.at[g], idx_v.at[b], sems[b])

        def consume(k, b):
            g = wid + k * _NW

            @pl.when(g < _SPLIT_BLK)
            def _():
                pltpu.make_async_copy(af_hbm.at[pl.ds(g * _B, _B)],
                                      rows_v.at[b], sems[b]).wait()
                pltpu.make_async_copy(mem_hbm.at[g], idx_v.at[b],
                                      sems[b]).wait()
                pltpu.sync_copy(rows_v.at[b], acc_sh.at[idx_v.at[b, 0]],
                                add=True)

            issue(k + _NRING, b)

        for _b in range(_NRING - 1):
            issue(_b, _b)

        zero16 = jnp.zeros((16,), jnp.float32)

        def zbody(i, carry):
            for j in range(FEAT // 16):
                rows_v[_NRING - 1, i, pl.ds(j * 16, 16)] = zero16
            return carry

        lax.fori_loop(0, _B, zbody, 0)

        zb = _NRING - 1
        off = 0
        for n in _WCH:
            pltpu.async_copy(rslice(zb, n), acc_sh.at[pl.ds(r0 + off, n)],
                             sems[zb])
            off += n

        @pl.when(last)
        def _():
            pltpu.async_copy(rslice(zb, 16), acc_sh.at[pl.ds(r0 + 624, 16)],
                             sems[zb])

        off = 0
        for n in _WCH:
            pltpu.make_async_copy(rslice(zb, n),
                                  acc_sh.at[pl.ds(r0 + off, n)],
                                  sems[zb]).wait()
            off += n

        @pl.when(last)
        def _():
            pltpu.make_async_copy(rslice(zb, 16),
                                  acc_sh.at[pl.ds(r0 + 624, 16)],
                                  sems[zb]).wait()

        issue(zb, zb)
        plsc.subcore_barrier()

        def tri(kk, carry):
            k = kk * _NRING
            for _b in range(_NRING):
                consume(k + _b, _b)
            return carry

        lax.fori_loop(0, _NTRI, tri, 0)
        plsc.subcore_barrier()

        def st_dsc(z, n):
            b = z % 2
            return (rslice(b, n),
                    out_hbm.at[pl.ds(ob + z * _B, n)], sems[b])

        for z, n in enumerate(_WCH):
            if z >= 2:
                src, dst, sem = st_dsc(z - 2, _WCH[z - 2])
                pltpu.make_async_copy(src, dst, sem).wait()
            src, dst, sem = st_dsc(z, n)
            pltpu.sync_copy(acc_sh.at[pl.ds(r0 + z * _B, n)], rslice(z % 2, n))
            pltpu.async_copy(src, dst, sem)
        for z in (len(_WCH) - 2, len(_WCH) - 1):
            src, dst, sem = st_dsc(z, _WCH[z])
            pltpu.make_async_copy(src, dst, sem).wait()

        @pl.when(last)
        def _():
            pltpu.sync_copy(acc_sh.at[pl.ds(r0 + 624, 16)], rslice(0, 16))
            pltpu.sync_copy(rslice(0, 16), out_hbm.at[pl.ds(ob + 624, 16)])

    return segsum


_segsum = _make_segsum()


def _tc_body(lo_ref, hi_ref, af_ref, mem_ref, out_ref, rows_s, mem_s,
             sem0, sem1):
    w = pl.program_id(0)
    lo = lo_ref[w]
    hi = hi_ref[w]
    n = hi - lo
    out_ref[...] = jnp.zeros((_W, FEAT), jnp.float32)
    iot = lax.broadcasted_iota(jnp.int32, (_W, _CHK), 0) + w * _W
    sems = (sem0, sem1)

    def dscs(j, b):
        c = lo + j
        return (pltpu.make_async_copy(af_ref.at[pl.ds(c * _CHK, _CHK)],
                                      rows_s.at[b], sems[b]),
                pltpu.make_async_copy(mem_ref.at[c], mem_s.at[b], sems[b]))

    def issue(j, b, guard):
        @pl.when(guard)
        def _():
            cp_r, cp_m = dscs(j, b)
            cp_r.start()
            cp_m.start()

    def step(j, b, guard):
        @pl.when(guard)
        def _():
            cp_r, cp_m = dscs(j, b)
            cp_r.wait()
            cp_m.wait()
            oh = (iot == mem_s[b]).astype(jnp.bfloat16)
            rows = rows_s[b]
            r_hi = rows.astype(jnp.bfloat16)
            r_lo = (rows - r_hi.astype(jnp.float32)).astype(jnp.bfloat16)
            dn = (((1,), (0,)), ((), ()))
            out_ref[...] += (
                lax.dot_general(oh, r_hi, dn,
                                preferred_element_type=jnp.float32)
                + lax.dot_general(oh, r_lo, dn,
                                  preferred_element_type=jnp.float32))

    issue(0, 0, n > 0)

    def pair(kk, carry):
        k = kk * 2
        issue(k + 1, 1, k + 1 < n)
        step(k, 0, k < n)
        issue(k + 2, 0, k + 2 < n)
        step(k + 1, 1, k + 1 < n)
        return carry

    lax.fori_loop(0, (n + 1) // 2, pair, 0)


def _tc_segsum(atom_features, mem3, lo_c, hi_c):
    grid_spec = pltpu.PrefetchScalarGridSpec(
        num_scalar_prefetch=2,
        grid=(_NWIN,),
        in_specs=[
            pl.BlockSpec(memory_space=pl.ANY),
            pl.BlockSpec(memory_space=pl.ANY),
        ],
        out_specs=pl.BlockSpec((_W, FEAT), lambda i, lo, hi: (i, 0)),
        scratch_shapes=[
            pltpu.VMEM((2, _CHK, FEAT), jnp.float32),
            pltpu.VMEM((2, 1, _CHK), jnp.int32),
            pltpu.SemaphoreType.DMA,
            pltpu.SemaphoreType.DMA,
        ],
    )
    return pl.pallas_call(
        _tc_body,
        grid_spec=grid_spec,
        out_shape=jax.ShapeDtypeStruct((_NWIN * _W, FEAT), jnp.float32),
    )(lo_c, hi_c, atom_features, mem3)


_MLP_BLK = 1000
_MLP_GRID = N_GRAPHS // _MLP_BLK


def _mlp_body(*refs):
    if len(refs) == 8:
        p0_ref, p1_ref, p2_ref, w1_ref, b1_ref, w2_ref, b2_ref, o_ref = refs
        g = p0_ref[...] + p1_ref[...] + p2_ref[...]
    else:
        p0_ref, p1_ref, w1_ref, b1_ref, w2_ref, b2_ref, o_ref = refs
        g = p0_ref[...] + p1_ref[...]
    h = jnp.dot(g, w1_ref[...], preferred_element_type=jnp.float32)
    h = jnp.maximum(h + b1_ref[...], 0.0)
    o = jnp.dot(h, w2_ref[...], preferred_element_type=jnp.float32)
    o_ref[...] = jnp.maximum(o + b2_ref[...], 0.0)


def _mlp(partials, p_tc, W1, b1, W2, b2):
    in_specs = [
        pl.BlockSpec((_MLP_BLK, FEAT), lambda i: (i, 0)),
        pl.BlockSpec((_MLP_BLK, FEAT), lambda i: (i + _MLP_GRID, 0)),
    ]
    args = [partials, partials]
    if p_tc is not None:
        in_specs.append(pl.BlockSpec((_MLP_BLK, FEAT), lambda i: (i, 0)))
        args.append(p_tc)
    in_specs += [
        pl.BlockSpec((FEAT, HIDDEN), lambda i: (0, 0)),
        pl.BlockSpec((1, HIDDEN), lambda i: (0, 0)),
        pl.BlockSpec((HIDDEN, FEAT), lambda i: (0, 0)),
        pl.BlockSpec((1, FEAT), lambda i: (0, 0)),
    ]
    args += [W1, b1.reshape(1, HIDDEN), W2, b2.reshape(1, FEAT)]
    return pl.pallas_call(
        _mlp_body,
        grid=(_MLP_GRID,),
        in_specs=in_specs,
        out_specs=pl.BlockSpec((_MLP_BLK, FEAT), lambda i: (i, 0)),
        out_shape=jax.ShapeDtypeStruct((N_GRAPHS, FEAT), jnp.float32),
    )(*args)


def kernel(atom_features, membership, W1, b1, W2, b2):
    mem_i32 = membership.astype(jnp.int32)
    mem_sc = mem_i32.reshape(_NB, 1, _B)

    if _SPLIT_BLK < _NB:
        mem_tc = mem_i32.reshape(_NCHK, 1, _CHK)
        seg_bounds = jnp.arange(_NWIN + 1, dtype=jnp.int32) * _W
        pos = jnp.searchsorted(mem_i32, seg_bounds,
                               side="left").astype(jnp.int32)
        lo_c = jnp.maximum(pos[:-1] // _CHK, _SPLIT_CHUNK)
        hi_c = jnp.minimum((pos[1:] + _CHK - 1) // _CHK, _NCHK)
        hi_c = jnp.maximum(hi_c, lo_c)
        partials = _segsum(atom_features, mem_sc)
        p_tc = _tc_segsum(atom_features, mem_tc, lo_c, hi_c)
    else:
        partials = _segsum(atom_features, mem_sc)
        p_tc = None

    return _mlp(partials, p_tc, W1, b1, W2, b2)
